# Initial kernel scaffold; baseline (speedup 1.0000x reference)
import jax
import jax.numpy as jnp
from jax.experimental import pallas as pl


def kernel(x, edge_index, W1a, b1a, W2a, b2a, W1b, b1b, W2b, b2b, W1c, b1c, W2c, b2c):
    raise NotImplementedError("write your pallas kernel here")


# speedup vs baseline: 5.5362x; 5.5362x over previous
import functools

import jax
import jax.numpy as jnp
from jax import lax
from jax.experimental import pallas as pl
from jax.experimental.pallas import tpu as pltpu
from jax.experimental.pallas import tpu_sc as plsc

N = 10000
E = 320000
D = 128

NC = 2
NS = 16
NW = NC * NS

CB = 128
NCHUNK = E // CB
CHUNKS_PER_W = -(-NCHUNK // NW)
NP = 10240
RPS = NP // NS

_mesh = plsc.VectorSubcoreMesh(core_axis_name="c", subcore_axis_name="s")


@functools.partial(
    pl.kernel,
    mesh=_mesh,
    out_type=jax.ShapeDtypeStruct((NC * NP, D), jnp.float32),
    scratch_types=[
        pltpu.VMEM((1, CB), jnp.int32),
        pltpu.VMEM((1, CB), jnp.int32),
        pltpu.VMEM((CB, D), jnp.float32),
        pltpu.VMEM_SHARED((NP, D), jnp.float32),
    ],
)
def _sc_aggregate(h_hbm, src_hbm, dst_hbm, zeros_hbm, out_hbm,
                  src_v, dst_v, rows_v, agg_sh):
    cid = lax.axis_index("c")
    sid = lax.axis_index("s")
    wid = sid * NC + cid

    pltpu.sync_copy(zeros_hbm, agg_sh.at[pl.ds(sid * RPS, RPS)])
    plsc.subcore_barrier()

    @pl.loop(0, CHUNKS_PER_W)
    def _(j):
        g = j * NW + wid

        @pl.when(g < NCHUNK)
        def _():
            pltpu.sync_copy(src_hbm.at[pl.ds(g, 1)], src_v)
            pltpu.sync_copy(dst_hbm.at[pl.ds(g, 1)], dst_v)
            pltpu.sync_copy(h_hbm.at[src_v.at[0]], rows_v)
            pltpu.sync_copy(rows_v, agg_sh.at[dst_v.at[0]], add=True)

    plsc.subcore_barrier()

    pltpu.sync_copy(agg_sh.at[pl.ds(sid * RPS, RPS)],
                    out_hbm.at[pl.ds(cid * NP + sid * RPS, RPS)])


def _mlp_body(h_ref, p0_ref, p1_ref, w1_ref, b1_ref, w2_ref, b2_ref, o_ref):
    z = h_ref[...] + p0_ref[...] + p1_ref[...]
    dn = (((1,), (0,)), ((), ()))
    a = lax.dot_general(z, w1_ref[...], dn,
                        precision=lax.Precision.HIGHEST,
                        preferred_element_type=jnp.float32)
    a = jnp.maximum(a + b1_ref[...], 0.0)
    o = lax.dot_general(a, w2_ref[...], dn,
                        precision=lax.Precision.HIGHEST,
                        preferred_element_type=jnp.float32)
    o_ref[...] = o + b2_ref[...]


def _tc_mlp(h, p0, p1, W1, b1, W2, b2):
    n, din = h.shape
    hmid = W1.shape[1]
    dout = W2.shape[1]
    bn = 1000
    grid = (n // bn,)
    return pl.pallas_call(
        _mlp_body,
        grid=grid,
        in_specs=[
            pl.BlockSpec((bn, din), lambda i: (i, 0)),
            pl.BlockSpec((bn, din), lambda i: (i, 0)),
            pl.BlockSpec((bn, din), lambda i: (i, 0)),
            pl.BlockSpec((din, hmid), lambda i: (0, 0)),
            pl.BlockSpec((1, hmid), lambda i: (0, 0)),
            pl.BlockSpec((hmid, dout), lambda i: (0, 0)),
            pl.BlockSpec((1, dout), lambda i: (0, 0)),
        ],
        out_specs=pl.BlockSpec((bn, dout), lambda i: (i, 0)),
        out_shape=jax.ShapeDtypeStruct((n, dout), jnp.float32),
    )(h, p0, p1, W1, b1.reshape(1, -1), W2, b2.reshape(1, -1))


def kernel(x, edge_index, W1a, b1a, W2a, b2a, W1b, b1b, W2b, b2b,
           W1c, b1c, W2c, b2c):
    src2d = edge_index[0].reshape(NCHUNK, CB)
    dst2d = edge_index[1].reshape(NCHUNK, CB)
    zeros = jnp.zeros((RPS, D), jnp.float32)

    h = x
    for W1, b1, W2, b2 in ((W1a, b1a, W2a, b2a),
                           (W1b, b1b, W2b, b2b),
                           (W1c, b1c, W2c, b2c)):
        parts = _sc_aggregate(h, src2d, dst2d, zeros)
        h = _tc_mlp(h, parts[:N], parts[NP:NP + N], W1, b1, W2, b2)
    return h


# speedup vs baseline: 8.1646x; 1.4748x over previous
import functools

import jax
import jax.numpy as jnp
from jax import lax
from jax.experimental import pallas as pl
from jax.experimental.pallas import tpu as pltpu
from jax.experimental.pallas import tpu_sc as plsc

N = 10000
E = 320000
D = 128

NC = 2
NS = 16
NW = NC * NS

CB = 128
NCHUNK = E // CB
CHUNKS_PER_W = -(-NCHUNK // NW)
NP = 10240
RPS = NP // NS

_mesh = plsc.VectorSubcoreMesh(core_axis_name="c", subcore_axis_name="s")


@functools.partial(
    pl.kernel,
    mesh=_mesh,
    out_type=jax.ShapeDtypeStruct((NC * NP, D), jnp.float32),
    scratch_types=[
        pltpu.VMEM((2, 1, CB), jnp.int32),
        pltpu.VMEM((2, 1, CB), jnp.int32),
        pltpu.VMEM((2, CB, D), jnp.float32),
        pltpu.VMEM_SHARED((NP, D), jnp.float32),
        pltpu.SemaphoreType.DMA,
        pltpu.SemaphoreType.DMA,
    ],
)
def _sc_aggregate(h_hbm, src_hbm, dst_hbm, zeros_hbm, out_hbm,
                  src_v, dst_v, rows_v, agg_sh, sem0, sem1):
    cid = lax.axis_index("c")
    sid = lax.axis_index("s")
    wid = sid * NC + cid
    sems = (sem0, sem1)

    pltpu.sync_copy(zeros_hbm, agg_sh.at[pl.ds(sid * RPS, RPS)])
    plsc.subcore_barrier()

    for b in (0, 1):
        g = b * NW + wid
        pltpu.sync_copy(src_hbm.at[pl.ds(g, 1)], src_v.at[b])
        pltpu.sync_copy(dst_hbm.at[pl.ds(g, 1)], dst_v.at[b])
        pltpu.async_copy(h_hbm.at[src_v.at[b].at[0]], rows_v.at[b], sems[b])

    @pl.loop(0, CHUNKS_PER_W + 1, step=2)
    def _(j):
        for b in (0, 1):
            g = (j + b) * NW + wid
            gn = (j + b + 2) * NW + wid

            @pl.when(g < NCHUNK)
            def _():
                pltpu.make_async_copy(
                    h_hbm.at[src_v.at[b].at[0]], rows_v.at[b], sems[b]).wait()
                pltpu.sync_copy(rows_v.at[b],
                                agg_sh.at[dst_v.at[b].at[0]], add=True)

            @pl.when(gn < NCHUNK)
            def _():
                pltpu.sync_copy(src_hbm.at[pl.ds(gn, 1)], src_v.at[b])
                pltpu.sync_copy(dst_hbm.at[pl.ds(gn, 1)], dst_v.at[b])
                pltpu.async_copy(h_hbm.at[src_v.at[b].at[0]],
                                 rows_v.at[b], sems[b])

    plsc.subcore_barrier()

    pltpu.sync_copy(agg_sh.at[pl.ds(sid * RPS, RPS)],
                    out_hbm.at[pl.ds(cid * NP + sid * RPS, RPS)])


def _mlp_body(h_ref, p0_ref, p1_ref, w1_ref, b1_ref, w2_ref, b2_ref, o_ref):
    z = h_ref[...] + p0_ref[...] + p1_ref[...]
    dn = (((1,), (0,)), ((), ()))
    a = lax.dot_general(z, w1_ref[...], dn,
                        precision=lax.Precision.HIGHEST,
                        preferred_element_type=jnp.float32)
    a = jnp.maximum(a + b1_ref[...], 0.0)
    o = lax.dot_general(a, w2_ref[...], dn,
                        precision=lax.Precision.HIGHEST,
                        preferred_element_type=jnp.float32)
    o_ref[...] = o + b2_ref[...]


def _tc_mlp(h, p0, p1, W1, b1, W2, b2):
    n, din = h.shape
    hmid = W1.shape[1]
    dout = W2.shape[1]
    bn = 1000
    grid = (n // bn,)
    return pl.pallas_call(
        _mlp_body,
        grid=grid,
        in_specs=[
            pl.BlockSpec((bn, din), lambda i: (i, 0)),
            pl.BlockSpec((bn, din), lambda i: (i, 0)),
            pl.BlockSpec((bn, din), lambda i: (i, 0)),
            pl.BlockSpec((din, hmid), lambda i: (0, 0)),
            pl.BlockSpec((1, hmid), lambda i: (0, 0)),
            pl.BlockSpec((hmid, dout), lambda i: (0, 0)),
            pl.BlockSpec((1, dout), lambda i: (0, 0)),
        ],
        out_specs=pl.BlockSpec((bn, dout), lambda i: (i, 0)),
        out_shape=jax.ShapeDtypeStruct((n, dout), jnp.float32),
    )(h, p0, p1, W1, b1.reshape(1, -1), W2, b2.reshape(1, -1))


def kernel(x, edge_index, W1a, b1a, W2a, b2a, W1b, b1b, W2b, b2b,
           W1c, b1c, W2c, b2c):
    src2d = edge_index[0].reshape(NCHUNK, CB)
    dst2d = edge_index[1].reshape(NCHUNK, CB)
    zeros = jnp.zeros((RPS, D), jnp.float32)

    h = x
    for W1, b1, W2, b2 in ((W1a, b1a, W2a, b2a),
                           (W1b, b1b, W2b, b2b),
                           (W1c, b1c, W2c, b2c)):
        parts = _sc_aggregate(h, src2d, dst2d, zeros)
        h = _tc_mlp(h, parts[:N], parts[NP:NP + N], W1, b1, W2, b2)
    return h
